# Initial kernel scaffold; baseline (speedup 1.0000x reference)
import jax
import jax.numpy as jnp
from jax.experimental import pallas as pl


def kernel(pos, forces, batch, edge_index, beam_col, W_e1, b_e1, W_e12, b_e12, W_h1, b_h1, W_h12, b_h12, W_geom, b_geom, gn_w, gn_b, gn_ms, W_linh, b_linh, W_other, b_other, W_disp1, b_disp1, W_disp2, b_disp2, W_N1, b_N1, W_N2, b_N2, W_M1, b_M1, W_M2, b_M2):
    raise NotImplementedError("write your pallas kernel here")


# speedup vs baseline: 2.4724x; 2.4724x over previous
import functools

import jax
import jax.numpy as jnp
from jax import lax
from jax.experimental import pallas as pl
from jax.experimental.pallas import tpu as pltpu
from jax.experimental.pallas import tpu_sc as plsc

N = 10000
E = 320000
H = 128
NB = 4
NC = 2
NS = 16
NW = NC * NS
EPW = E // NW
CH = 80
NCHUNK = EPW // CH
NGRP = N // 80
ETILE = 1600


def _swish(x):
    return x * jax.nn.sigmoid(x)


def _dot(a, b):
    return jnp.dot(a.astype(jnp.bfloat16), b.astype(jnp.bfloat16),
                   preferred_element_type=jnp.float32)


def _rb(x):
    return x.astype(jnp.bfloat16).astype(jnp.float32)


def _mesh():
    return plsc.VectorSubcoreMesh(
        core_axis_name="c", subcore_axis_name="s", num_cores=NC, num_subcores=NS)


def _sc_relpos_body(pos_hbm, npos_hbm, src_hbm, dst_hbm, out_hbm,
                    idx_s, idx_d, buf, sem):
    c = lax.axis_index("c")
    s = lax.axis_index("s")
    wid = c * NS + s

    def chunk(k, carry):
        base = wid * EPW + k * CH
        pltpu.sync_copy(src_hbm.at[pl.ds(base, CH)], idx_s)
        pltpu.sync_copy(dst_hbm.at[pl.ds(base, CH)], idx_d)
        pltpu.async_copy(pos_hbm.at[idx_s], buf, sem).wait()
        pltpu.async_copy(npos_hbm.at[idx_d], buf, sem, add=True).wait()
        pltpu.sync_copy(buf, out_hbm.at[pl.ds(base, CH)])
        return carry

    lax.fori_loop(0, NCHUNK, chunk, 0)


def _sc_relpos(wtab, nwtab, src, dst):
    fn = pl.kernel(
        _sc_relpos_body,
        out_type=jax.ShapeDtypeStruct((E, H), jnp.float32),
        mesh=_mesh(),
        scratch_types=[
            pltpu.VMEM((CH,), jnp.int32),
            pltpu.VMEM((CH,), jnp.int32),
            pltpu.VMEM((CH, H), jnp.float32),
            pltpu.SemaphoreType.DMA,
        ],
    )
    return fn(wtab, nwtab, src, dst)


def _sc_edge_body(ce_hbm, p_hbm, q_hbm, hh_hbm, src_hbm, dst_hbm, out_hbm,
                  idx_s, idx_d, buf_t, buf_h, agg, sem):
    c = lax.axis_index("c")
    s = lax.axis_index("s")
    wid = c * NS + s

    def zr(r, carry):
        z = jnp.zeros((16,), jnp.float32)
        for j in range(8):
            buf_t[r, pl.ds(j * 16, 16)] = z
        return carry

    lax.fori_loop(0, CH, zr, 0)
    ngroups = (NGRP - s + NS - 1) // NS

    def zgrp(k, carry):
        pltpu.sync_copy(buf_t, agg.at[pl.ds((s + NS * k) * CH, CH)])
        return carry

    lax.fori_loop(0, ngroups, zgrp, 0)
    plsc.subcore_barrier()

    def chunk(k, carry):
        base = wid * EPW + k * CH
        pltpu.sync_copy(src_hbm.at[pl.ds(base, CH)], idx_s)
        pltpu.sync_copy(dst_hbm.at[pl.ds(base, CH)], idx_d)
        pltpu.sync_copy(ce_hbm.at[pl.ds(base, CH)], buf_t)
        pltpu.async_copy(hh_hbm.at[idx_s], buf_h, sem).wait()
        pltpu.async_copy(p_hbm.at[idx_s], buf_t, sem, add=True).wait()
        pltpu.async_copy(q_hbm.at[idx_d], buf_t, sem, add=True).wait()

        def row(r, rc):
            for j in range(8):
                sl = pl.ds(j * 16, 16)
                t = buf_t[r, sl]
                hh = buf_h[r, sl]
                buf_t[r, sl] = hh * t / (1.0 + jnp.exp(-t))
            return rc

        lax.fori_loop(0, CH, row, 0)
        pltpu.sync_copy(buf_t, agg.at[idx_d], add=True)
        return carry

    lax.fori_loop(0, NCHUNK, chunk, 0)
    plsc.subcore_barrier()

    def dgrp(k, carry):
        row0 = (s + NS * k) * CH
        pltpu.sync_copy(agg.at[pl.ds(row0, CH)], buf_t)
        pltpu.sync_copy(buf_t, out_hbm.at[c, pl.ds(row0, CH)])
        return carry

    lax.fori_loop(0, ngroups, dgrp, 0)


def _sc_edge(ce, p, q, hh, src, dst):
    fn = pl.kernel(
        _sc_edge_body,
        out_type=jax.ShapeDtypeStruct((NC, N, H), jnp.float32),
        mesh=_mesh(),
        scratch_types=[
            pltpu.VMEM((CH,), jnp.int32),
            pltpu.VMEM((CH,), jnp.int32),
            pltpu.VMEM((CH, H), jnp.float32),
            pltpu.VMEM((CH, H), jnp.float32),
            pltpu.VMEM_SHARED((N, H), jnp.float32),
            pltpu.SemaphoreType.DMA,
        ],
    )
    return fn(ce, p, q, hh, src, dst)


def _tc_edge_embed_body(relp_ref, bc_ref, we1_ref, be1_ref, wbc_ref,
                        be12_ref, wE_ref, bg_ref, o0, o1, o2, o3):
    x = relp_ref[...]
    bc = bc_ref[...]
    ln = jnp.sqrt(jnp.sum(x * x, axis=1, keepdims=True))
    e1 = _dot(x[:, :16], we1_ref[...]) + be1_ref[...][None, :]
    zp = jnp.zeros((ETILE, 13), jnp.float32)
    ea = jnp.concatenate([bc[:, :2], ln, zp], axis=1)
    e2 = _dot(ea, wbc_ref[...]) + be12_ref[...][None, :]
    e = _swish(jnp.concatenate([e1, e2], axis=1))
    outs = (o0, o1, o2, o3)
    for i in range(NB):
        outs[i][...] = _dot(e, wE_ref[i]) + bg_ref[i][None, :]


def _tc_edge_embed(relp, bc_pad, we1, be1, wbc, be12, wE, bg):
    grid = (E // ETILE,)
    full = lambda a: pl.BlockSpec(a.shape, lambda i: (0,) * a.ndim)
    out_bs = pl.BlockSpec((ETILE, H), lambda i: (i, 0))
    return pl.pallas_call(
        _tc_edge_embed_body,
        grid=grid,
        in_specs=[
            pl.BlockSpec((ETILE, H), lambda i: (i, 0)),
            pl.BlockSpec((ETILE, 8), lambda i: (i, 0)),
            full(we1), full(be1), full(wbc), full(be12),
            full(wE), full(bg),
        ],
        out_specs=[out_bs, out_bs, out_bs, out_bs],
        out_shape=[jax.ShapeDtypeStruct((E, H), jnp.float32)] * NB,
    )(relp, bc_pad, we1, be1, wbc, be12, wE, bg)


NT = 2000


def _row_bs(width):
    return pl.BlockSpec((NT, width), lambda i: (i, 0))


def _full_bs(a):
    return pl.BlockSpec(a.shape, lambda i: (0,) * a.ndim)


def _tc_node0_body(f_ref, pos_ref, wh1_ref, bh1_ref, wh12_ref, bh12_ref,
                   wS_ref, wD_ref, we1_ref,
                   h_ref, p_ref, q_ref, w_ref, nw_ref):
    f = f_ref[...]
    fn = jnp.sqrt(jnp.sum(f * f, axis=1, keepdims=True))
    h1 = _dot(f, wh1_ref[...]) + bh1_ref[...][None, :]
    h2 = fn * wh12_ref[...] + bh12_ref[...][None, :]
    h = _swish(jnp.concatenate([h1, h2], axis=1))
    h_ref[...] = h
    p_ref[...] = _dot(h, wS_ref[...])
    q_ref[...] = _dot(h, wD_ref[...])
    pos = pos_ref[...]
    zpad = jnp.zeros((NT, 112), jnp.float32)
    w = jnp.concatenate([pos, zpad], axis=1)
    w_ref[...] = w
    nw_ref[...] = -w


def _tc_node0(f_pad, pos_pad, wh1p, bh1, wh12, bh12, wS, wD, we1p):
    return pl.pallas_call(
        _tc_node0_body,
        grid=(N // NT,),
        in_specs=[_row_bs(16), _row_bs(16), _full_bs(wh1p), _full_bs(bh1),
                  _full_bs(wh12), _full_bs(bh12), _full_bs(wS), _full_bs(wD),
                  _full_bs(we1p)],
        out_specs=[_row_bs(H)] * 5,
        out_shape=[jax.ShapeDtypeStruct((N, H), jnp.float32)] * 5,
    )(f_pad, pos_pad, wh1p, bh1, wh12, bh12, wS, wD, we1p)


def _tc_norm_body(parts_ref, gnw_ref, gnb_ref, gnms_ref, hn_ref):
    agg = parts_ref[0] + parts_ref[1]
    mean = jnp.mean(agg, axis=0, keepdims=True)
    cent = agg - gnms_ref[...][None, :] * mean
    var = jnp.mean(cent * cent, axis=0, keepdims=True)
    hn_ref[...] = _swish(gnw_ref[...][None, :] * cent / jnp.sqrt(var + 1e-5)
                         + gnb_ref[...][None, :])


def _tc_norm(parts, gnw, gnb, gnms):
    return pl.pallas_call(
        _tc_norm_body,
        out_shape=jax.ShapeDtypeStruct((N, H), jnp.float32),
    )(parts, gnw, gnb, gnms)


def _tc_update_body(hn_ref, wl_ref, bl_ref, wo_ref, bo_ref, wS_ref, wD_ref,
                    h_ref, p_ref, q_ref):
    h = _swish(_dot(hn_ref[...], wl_ref[...]) + bl_ref[...][None, :])
    h = _swish(_dot(h, wo_ref[...]) + bo_ref[...][None, :])
    h_ref[...] = h
    p_ref[...] = _dot(h, wS_ref[...])
    q_ref[...] = _dot(h, wD_ref[...])


def _tc_node(parts, gnw, gnb, gnms, wl, bl, wo, bo, wS, wD):
    hn = _tc_norm(parts, gnw, gnb, gnms)
    return pl.pallas_call(
        _tc_update_body,
        grid=(N // NT,),
        in_specs=[_row_bs(H), _full_bs(wl), _full_bs(bl), _full_bs(wo),
                  _full_bs(bo), _full_bs(wS), _full_bs(wD)],
        out_specs=[_row_bs(H)] * 3,
        out_shape=[jax.ShapeDtypeStruct((N, H), jnp.float32)] * 3,
    )(hn, wl, bl, wo, bo, wS, wD)


def _tc_final_body(hn_ref, wl_ref, bl_ref, wo_ref, bo_ref,
                   wd1_ref, bd1_ref, wd2_ref, bd2_ref,
                   wn1_ref, bn1_ref, wn2_ref, bn2_ref,
                   wm1_ref, bm1_ref, wm2_ref, bm2_ref,
                   disp_ref, n_ref, m_ref):
    h = _swish(_dot(hn_ref[...], wl_ref[...]) + bl_ref[...][None, :])
    h = _swish(_dot(h, wo_ref[...]) + bo_ref[...][None, :])
    disp_ref[...] = (_dot(_swish(_dot(h, wd1_ref[...]) + bd1_ref[...][None, :]),
                          wd2_ref[...]) + bd2_ref[...][None, :])
    n_ref[...] = (_dot(_swish(_dot(h, wn1_ref[...]) + bn1_ref[...][None, :]),
                       wn2_ref[...]) + bn2_ref[...][None, :])
    m_ref[...] = (_dot(_swish(_dot(h, wm1_ref[...]) + bm1_ref[...][None, :]),
                       wm2_ref[...]) + bm2_ref[...][None, :])


def _tc_final(parts, gnw, gnb, gnms, wl, bl, wo, bo,
              wd1, bd1, wd2, bd2, wn1, bn1, wn2, bn2, wm1, bm1, wm2, bm2):
    hn = _tc_norm(parts, gnw, gnb, gnms)
    ws = (wl, bl, wo, bo, wd1, bd1, wd2, bd2, wn1, bn1, wn2, bn2,
          wm1, bm1, wm2, bm2)
    return pl.pallas_call(
        _tc_final_body,
        grid=(N // NT,),
        in_specs=[_row_bs(H)] + [_full_bs(w) for w in ws],
        out_specs=[pl.BlockSpec((NT, 3), lambda i: (i, 0)),
                   pl.BlockSpec((NT, 18), lambda i: (i, 0)),
                   pl.BlockSpec((NT, 18), lambda i: (i, 0))],
        out_shape=[
            jax.ShapeDtypeStruct((N, 3), jnp.float32),
            jax.ShapeDtypeStruct((N, 18), jnp.float32),
            jax.ShapeDtypeStruct((N, 18), jnp.float32),
        ],
    )(hn, *ws)


def kernel(pos, forces, batch, edge_index, beam_col, W_e1, b_e1, W_e12, b_e12,
           W_h1, b_h1, W_h12, b_h12, W_geom, b_geom, gn_w, gn_b, gn_ms,
           W_linh, b_linh, W_other, b_other, W_disp1, b_disp1, W_disp2,
           b_disp2, W_N1, b_N1, W_N2, b_N2, W_M1, b_M1, W_M2, b_M2):
    src = edge_index[0]
    dst = edge_index[1]
    pos_pad = jnp.pad(pos, ((0, 0), (0, 13)))
    f_pad = jnp.pad(forces, ((0, 0), (0, 13)))
    bc_pad = jnp.pad(beam_col, ((0, 0), (0, 6)))
    we1p = jnp.pad(W_e1, ((0, 13), (0, 0)))
    wh1p = jnp.pad(W_h1, ((0, 13), (0, 0)))
    we12p = jnp.pad(W_e12, ((0, 13), (0, 0)))
    wE = W_geom[:, :H, :]
    wS = W_geom[:, H:2 * H, :]
    wD = W_geom[:, 2 * H:, :]

    h, p, q, wtab, nwtab = _tc_node0(f_pad, pos_pad, wh1p, b_h1, W_h12, b_h12,
                                     wS[0], wD[0], W_e1)
    relp = _sc_relpos(wtab, nwtab, src, dst)
    ces = _tc_edge_embed(relp, bc_pad, we1p, b_e1, we12p, b_e12, wE, b_geom)
    for i in range(NB):
        parts = _sc_edge(ces[i], p, q, h, src, dst)
        if i < NB - 1:
            h, p, q = _tc_node(parts, gn_w[i], gn_b[i], gn_ms[i],
                               W_linh[i], b_linh[i], W_other[i], b_other[i],
                               wS[i + 1], wD[i + 1])
    return _tc_final(parts, gn_w[3], gn_b[3], gn_ms[3],
                     W_linh[3], b_linh[3], W_other[3], b_other[3],
                     W_disp1, b_disp1, W_disp2, b_disp2,
                     W_N1, b_N1, W_N2, b_N2, W_M1, b_M1, W_M2, b_M2)


# speedup vs baseline: 3.1171x; 1.2608x over previous
import functools

import jax
import jax.numpy as jnp
from jax import lax
from jax.experimental import pallas as pl
from jax.experimental.pallas import tpu as pltpu
from jax.experimental.pallas import tpu_sc as plsc

N = 10000
E = 320000
H = 128
NB = 4
NC = 2
NS = 16
NW = NC * NS
EPW = E // NW
CH = 80
NCHUNK = EPW // CH
NGRP = N // 80
ETILE = 1600


def _swish(x):
    return x * jax.nn.sigmoid(x)


def _dot(a, b):
    return jnp.dot(a.astype(jnp.bfloat16), b.astype(jnp.bfloat16),
                   preferred_element_type=jnp.float32)


def _rb(x):
    return x.astype(jnp.bfloat16).astype(jnp.float32)


def _mesh():
    return plsc.VectorSubcoreMesh(
        core_axis_name="c", subcore_axis_name="s", num_cores=NC, num_subcores=NS)


def _sc_relpos_body(pos_hbm, npos_hbm, src_hbm, dst_hbm, out_hbm,
                    idx_s, idx_d, buf, sem):
    c = lax.axis_index("c")
    s = lax.axis_index("s")
    wid = c * NS + s

    def chunk(k, carry):
        base = wid * EPW + k * CH
        pltpu.sync_copy(src_hbm.at[pl.ds(base, CH)], idx_s)
        pltpu.sync_copy(dst_hbm.at[pl.ds(base, CH)], idx_d)
        pltpu.async_copy(pos_hbm.at[idx_s], buf, sem).wait()
        pltpu.async_copy(npos_hbm.at[idx_d], buf, sem, add=True).wait()
        pltpu.sync_copy(buf, out_hbm.at[pl.ds(base, CH)])
        return carry

    lax.fori_loop(0, NCHUNK, chunk, 0)


def _sc_relpos(wtab, nwtab, src, dst):
    fn = pl.kernel(
        _sc_relpos_body,
        out_type=jax.ShapeDtypeStruct((E, H), jnp.float32),
        mesh=_mesh(),
        scratch_types=[
            pltpu.VMEM((CH,), jnp.int32),
            pltpu.VMEM((CH,), jnp.int32),
            pltpu.VMEM((CH, H), jnp.float32),
            pltpu.SemaphoreType.DMA,
        ],
    )
    return fn(wtab, nwtab, src, dst)


def _sc_edge_body(ce_hbm, p_hbm, q_hbm, hh_hbm, src_hbm, dst_hbm, out_hbm,
                  idx_sa, idx_da, bta, bha, idx_sb, idx_db, btb, bhb,
                  agg, sem_ca, sem_ga, sem_cb, sem_gb):
    c = lax.axis_index("c")
    s = lax.axis_index("s")
    wid = c * NS + s

    def zr(r, carry):
        z = jnp.zeros((16,), jnp.float32)
        for j in range(8):
            bta[r, pl.ds(j * 16, 16)] = z
        return carry

    lax.fori_loop(0, CH, zr, 0)
    ngroups = (NGRP - s + NS - 1) // NS

    def zgrp(k, carry):
        pltpu.sync_copy(bta, agg.at[pl.ds((s + NS * k) * CH, CH)])
        return carry

    lax.fori_loop(0, ngroups, zgrp, 0)
    plsc.subcore_barrier()

    def load_idx(base, idx_s, idx_d):
        pltpu.sync_copy(src_hbm.at[pl.ds(base, CH)], idx_s)
        pltpu.sync_copy(dst_hbm.at[pl.ds(base, CH)], idx_d)

    def start_gathers(idx_s, idx_d, bt, bh, sem_g):
        d1 = pltpu.async_copy(hh_hbm.at[idx_s], bh, sem_g)
        d2 = pltpu.async_copy(p_hbm.at[idx_s], bt, sem_g, add=True)
        d3 = pltpu.async_copy(q_hbm.at[idx_d], bt, sem_g, add=True)
        return d1, d2, d3

    def compute_scatter(bt, bh, idx_d):
        def row(r, rc):
            for j in range(8):
                sl = pl.ds(j * 16, 16)
                t = bt[r, sl]
                hh = bh[r, sl]
                bt[r, sl] = hh * t / (1.0 + jnp.exp(-t))
            return rc

        lax.fori_loop(0, CH, row, 0)
        pltpu.sync_copy(bt, agg.at[idx_d], add=True)

    def pair(kp, carry):
        base_a = wid * EPW + (2 * kp) * CH
        base_b = base_a + CH
        load_idx(base_a, idx_sa, idx_da)
        load_idx(base_b, idx_sb, idx_db)
        ca = pltpu.async_copy(ce_hbm.at[pl.ds(base_a, CH)], bta, sem_ca)
        cb = pltpu.async_copy(ce_hbm.at[pl.ds(base_b, CH)], btb, sem_cb)
        ca.wait()
        da = start_gathers(idx_sa, idx_da, bta, bha, sem_ga)
        cb.wait()
        db = start_gathers(idx_sb, idx_db, btb, bhb, sem_gb)
        for dd in da:
            dd.wait()
        compute_scatter(bta, bha, idx_da)
        for dd in db:
            dd.wait()
        compute_scatter(btb, bhb, idx_db)
        return carry

    lax.fori_loop(0, NCHUNK // 2, pair, 0)
    base_t = wid * EPW + (NCHUNK - 1) * CH
    load_idx(base_t, idx_sa, idx_da)
    pltpu.sync_copy(ce_hbm.at[pl.ds(base_t, CH)], bta)
    for dd in start_gathers(idx_sa, idx_da, bta, bha, sem_ga):
        dd.wait()
    compute_scatter(bta, bha, idx_da)
    plsc.subcore_barrier()

    def dgrp(k, carry):
        row0 = (s + NS * k) * CH
        pltpu.sync_copy(agg.at[pl.ds(row0, CH)], bta)
        pltpu.sync_copy(bta, out_hbm.at[c, pl.ds(row0, CH)])
        return carry

    lax.fori_loop(0, ngroups, dgrp, 0)


def _sc_edge(ce, p, q, hh, src, dst):
    fn = pl.kernel(
        _sc_edge_body,
        out_type=jax.ShapeDtypeStruct((NC, N, H), jnp.float32),
        mesh=_mesh(),
        scratch_types=[
            pltpu.VMEM((CH,), jnp.int32),
            pltpu.VMEM((CH,), jnp.int32),
            pltpu.VMEM((CH, H), jnp.float32),
            pltpu.VMEM((CH, H), jnp.float32),
            pltpu.VMEM((CH,), jnp.int32),
            pltpu.VMEM((CH,), jnp.int32),
            pltpu.VMEM((CH, H), jnp.float32),
            pltpu.VMEM((CH, H), jnp.float32),
            pltpu.VMEM_SHARED((N, H), jnp.float32),
            pltpu.SemaphoreType.DMA,
            pltpu.SemaphoreType.DMA,
            pltpu.SemaphoreType.DMA,
            pltpu.SemaphoreType.DMA,
        ],
    )
    return fn(ce, p, q, hh, src, dst)


def _tc_edge_embed_body(relp_ref, bc_ref, we1_ref, be1_ref, wbc_ref,
                        be12_ref, wE_ref, bg_ref, o0, o1, o2, o3):
    x = relp_ref[...]
    bc = bc_ref[...]
    ln = jnp.sqrt(jnp.sum(x * x, axis=1, keepdims=True))
    e1 = _dot(x[:, :16], we1_ref[...]) + be1_ref[...][None, :]
    zp = jnp.zeros((ETILE, 13), jnp.float32)
    ea = jnp.concatenate([bc[:, :2], ln, zp], axis=1)
    e2 = _dot(ea, wbc_ref[...]) + be12_ref[...][None, :]
    e = _swish(jnp.concatenate([e1, e2], axis=1))
    outs = (o0, o1, o2, o3)
    for i in range(NB):
        outs[i][...] = _dot(e, wE_ref[i]) + bg_ref[i][None, :]


def _tc_edge_embed(relp, bc_pad, we1, be1, wbc, be12, wE, bg):
    grid = (E // ETILE,)
    full = lambda a: pl.BlockSpec(a.shape, lambda i: (0,) * a.ndim)
    out_bs = pl.BlockSpec((ETILE, H), lambda i: (i, 0))
    return pl.pallas_call(
        _tc_edge_embed_body,
        grid=grid,
        in_specs=[
            pl.BlockSpec((ETILE, H), lambda i: (i, 0)),
            pl.BlockSpec((ETILE, 8), lambda i: (i, 0)),
            full(we1), full(be1), full(wbc), full(be12),
            full(wE), full(bg),
        ],
        out_specs=[out_bs, out_bs, out_bs, out_bs],
        out_shape=[jax.ShapeDtypeStruct((E, H), jnp.float32)] * NB,
    )(relp, bc_pad, we1, be1, wbc, be12, wE, bg)


NT = 2000


def _row_bs(width):
    return pl.BlockSpec((NT, width), lambda i: (i, 0))


def _full_bs(a):
    return pl.BlockSpec(a.shape, lambda i: (0,) * a.ndim)


def _tc_node0_body(f_ref, pos_ref, wh1_ref, bh1_ref, wh12_ref, bh12_ref,
                   wS_ref, wD_ref, we1_ref,
                   h_ref, p_ref, q_ref, w_ref, nw_ref):
    f = f_ref[...]
    fn = jnp.sqrt(jnp.sum(f * f, axis=1, keepdims=True))
    h1 = _dot(f, wh1_ref[...]) + bh1_ref[...][None, :]
    h2 = fn * wh12_ref[...] + bh12_ref[...][None, :]
    h = _swish(jnp.concatenate([h1, h2], axis=1))
    h_ref[...] = h
    p_ref[...] = _dot(h, wS_ref[...])
    q_ref[...] = _dot(h, wD_ref[...])
    pos = pos_ref[...]
    zpad = jnp.zeros((NT, 112), jnp.float32)
    w = jnp.concatenate([pos, zpad], axis=1)
    w_ref[...] = w
    nw_ref[...] = -w


def _tc_node0(f_pad, pos_pad, wh1p, bh1, wh12, bh12, wS, wD, we1p):
    return pl.pallas_call(
        _tc_node0_body,
        grid=(N // NT,),
        in_specs=[_row_bs(16), _row_bs(16), _full_bs(wh1p), _full_bs(bh1),
                  _full_bs(wh12), _full_bs(bh12), _full_bs(wS), _full_bs(wD),
                  _full_bs(we1p)],
        out_specs=[_row_bs(H)] * 5,
        out_shape=[jax.ShapeDtypeStruct((N, H), jnp.float32)] * 5,
    )(f_pad, pos_pad, wh1p, bh1, wh12, bh12, wS, wD, we1p)


def _tc_norm_body(parts_ref, gnw_ref, gnb_ref, gnms_ref, hn_ref):
    agg = parts_ref[0] + parts_ref[1]
    mean = jnp.mean(agg, axis=0, keepdims=True)
    cent = agg - gnms_ref[...][None, :] * mean
    var = jnp.mean(cent * cent, axis=0, keepdims=True)
    hn_ref[...] = _swish(gnw_ref[...][None, :] * cent / jnp.sqrt(var + 1e-5)
                         + gnb_ref[...][None, :])


def _tc_norm(parts, gnw, gnb, gnms):
    return pl.pallas_call(
        _tc_norm_body,
        out_shape=jax.ShapeDtypeStruct((N, H), jnp.float32),
    )(parts, gnw, gnb, gnms)


def _tc_update_body(hn_ref, wl_ref, bl_ref, wo_ref, bo_ref, wS_ref, wD_ref,
                    h_ref, p_ref, q_ref):
    h = _swish(_dot(hn_ref[...], wl_ref[...]) + bl_ref[...][None, :])
    h = _swish(_dot(h, wo_ref[...]) + bo_ref[...][None, :])
    h_ref[...] = h
    p_ref[...] = _dot(h, wS_ref[...])
    q_ref[...] = _dot(h, wD_ref[...])


def _tc_node(parts, gnw, gnb, gnms, wl, bl, wo, bo, wS, wD):
    hn = _tc_norm(parts, gnw, gnb, gnms)
    return pl.pallas_call(
        _tc_update_body,
        grid=(N // NT,),
        in_specs=[_row_bs(H), _full_bs(wl), _full_bs(bl), _full_bs(wo),
                  _full_bs(bo), _full_bs(wS), _full_bs(wD)],
        out_specs=[_row_bs(H)] * 3,
        out_shape=[jax.ShapeDtypeStruct((N, H), jnp.float32)] * 3,
    )(hn, wl, bl, wo, bo, wS, wD)


def _tc_final_body(hn_ref, wl_ref, bl_ref, wo_ref, bo_ref,
                   wd1_ref, bd1_ref, wd2_ref, bd2_ref,
                   wn1_ref, bn1_ref, wn2_ref, bn2_ref,
                   wm1_ref, bm1_ref, wm2_ref, bm2_ref,
                   disp_ref, n_ref, m_ref):
    h = _swish(_dot(hn_ref[...], wl_ref[...]) + bl_ref[...][None, :])
    h = _swish(_dot(h, wo_ref[...]) + bo_ref[...][None, :])
    disp_ref[...] = (_dot(_swish(_dot(h, wd1_ref[...]) + bd1_ref[...][None, :]),
                          wd2_ref[...]) + bd2_ref[...][None, :])
    n_ref[...] = (_dot(_swish(_dot(h, wn1_ref[...]) + bn1_ref[...][None, :]),
                       wn2_ref[...]) + bn2_ref[...][None, :])
    m_ref[...] = (_dot(_swish(_dot(h, wm1_ref[...]) + bm1_ref[...][None, :]),
                       wm2_ref[...]) + bm2_ref[...][None, :])


def _tc_final(parts, gnw, gnb, gnms, wl, bl, wo, bo,
              wd1, bd1, wd2, bd2, wn1, bn1, wn2, bn2, wm1, bm1, wm2, bm2):
    hn = _tc_norm(parts, gnw, gnb, gnms)
    ws = (wl, bl, wo, bo, wd1, bd1, wd2, bd2, wn1, bn1, wn2, bn2,
          wm1, bm1, wm2, bm2)
    return pl.pallas_call(
        _tc_final_body,
        grid=(N // NT,),
        in_specs=[_row_bs(H)] + [_full_bs(w) for w in ws],
        out_specs=[pl.BlockSpec((NT, 3), lambda i: (i, 0)),
                   pl.BlockSpec((NT, 18), lambda i: (i, 0)),
                   pl.BlockSpec((NT, 18), lambda i: (i, 0))],
        out_shape=[
            jax.ShapeDtypeStruct((N, 3), jnp.float32),
            jax.ShapeDtypeStruct((N, 18), jnp.float32),
            jax.ShapeDtypeStruct((N, 18), jnp.float32),
        ],
    )(hn, *ws)


def kernel(pos, forces, batch, edge_index, beam_col, W_e1, b_e1, W_e12, b_e12,
           W_h1, b_h1, W_h12, b_h12, W_geom, b_geom, gn_w, gn_b, gn_ms,
           W_linh, b_linh, W_other, b_other, W_disp1, b_disp1, W_disp2,
           b_disp2, W_N1, b_N1, W_N2, b_N2, W_M1, b_M1, W_M2, b_M2):
    src = edge_index[0]
    dst = edge_index[1]
    pos_pad = jnp.pad(pos, ((0, 0), (0, 13)))
    f_pad = jnp.pad(forces, ((0, 0), (0, 13)))
    bc_pad = jnp.pad(beam_col, ((0, 0), (0, 6)))
    we1p = jnp.pad(W_e1, ((0, 13), (0, 0)))
    wh1p = jnp.pad(W_h1, ((0, 13), (0, 0)))
    we12p = jnp.pad(W_e12, ((0, 13), (0, 0)))
    wE = W_geom[:, :H, :]
    wS = W_geom[:, H:2 * H, :]
    wD = W_geom[:, 2 * H:, :]

    h, p, q, wtab, nwtab = _tc_node0(f_pad, pos_pad, wh1p, b_h1, W_h12, b_h12,
                                     wS[0], wD[0], W_e1)
    relp = _sc_relpos(wtab, nwtab, src, dst)
    ces = _tc_edge_embed(relp, bc_pad, we1p, b_e1, we12p, b_e12, wE, b_geom)
    for i in range(NB):
        parts = _sc_edge(ces[i], p, q, h, src, dst)
        if i < NB - 1:
            h, p, q = _tc_node(parts, gn_w[i], gn_b[i], gn_ms[i],
                               W_linh[i], b_linh[i], W_other[i], b_other[i],
                               wS[i + 1], wD[i + 1])
    return _tc_final(parts, gn_w[3], gn_b[3], gn_ms[3],
                     W_linh[3], b_linh[3], W_other[3], b_other[3],
                     W_disp1, b_disp1, W_disp2, b_disp2,
                     W_N1, b_N1, W_N2, b_N2, W_M1, b_M1, W_M2, b_M2)
